# baseline (speedup 1.0000x reference)
import functools

import jax
import jax.numpy as jnp
from jax import lax
from jax.experimental import pallas as pl
from jax.experimental.pallas import tpu as pltpu
from jax.experimental.pallas import tpu_sc as plsc

NC = 2
NS = 16
NW = NC * NS
LN = 16
SB = 4
PF = 3


def _pack_table(weights_t):
    D, V = weights_t.shape
    nblk = (V + 127) // 128
    nfull = V // 128

    @functools.partial(
        pl.kernel,
        mesh=plsc.VectorSubcoreMesh(core_axis_name="c", subcore_axis_name="s"),
        out_type=jax.ShapeDtypeStruct((V // 2, 128), jnp.float32),
        compiler_params=pltpu.CompilerParams(use_tc_tiling_on_sc=True,
                                             needs_layout_passes=False),
        scratch_types=[
            pltpu.VMEM((2, D, 128), jnp.float32),
            pltpu.VMEM((2, D, 128), jnp.float32),
            [pltpu.SemaphoreType.DMA] * 2,
            [pltpu.SemaphoreType.DMA] * 2,
        ],
    )
    def pack_k(w_hbm, out_hbm, in_v, out_v, isem, osem):
        wid = lax.axis_index("s") * NC + lax.axis_index("c")
        iota = lax.iota(jnp.int32, LN)
        rvecs = [(iota + r) % LN for r in range(LN)]
        half = (iota % 2) * D
        niter = (nblk + NW - 1) // NW

        def fire_read(bb, slot):
            pltpu.async_copy(w_hbm.at[:, pl.ds(bb * 128, 128)], in_v.at[slot],
                             isem[slot])

        def wait_read(slot):
            pltpu.make_async_copy(w_hbm.at[:, pl.ds(0, 128)], in_v.at[slot],
                                  isem[slot]).wait()

        def transpose(slot, nbt):
            src = in_v.at[slot]
            dst = out_v.at[slot]

            @plsc.parallel_loop(0, nbt, unroll=2)
            def bt_body(bt):
                vc = iota + bt * LN
                row = lax.shift_right_logical(vc, 1)
                for k in range(D // LN):
                    feats = [rvecs[r] + (k * LN) for r in range(LN)]
                    vals = [plsc.load_gather(src, [feats[r], vc])
                            for r in range(LN)]
                    for r in range(LN):
                        plsc.store_scatter(dst, [row, half + feats[r]],
                                           vals[r])

        def step(i, slot):
            bb = i * NW + wid

            nxt = bb + NW
            @pl.when(nxt < nblk)
            def _():
                fire_read(nxt, (slot + 1) % 2)

            @pl.when((i >= 2) & (bb - 2 * NW < nfull))
            def _():
                pltpu.make_async_copy(out_v.at[slot, pl.ds(0, 64)],
                                      out_hbm.at[pl.ds(0, 64)],
                                      osem[slot]).wait()

            @pl.when(bb < nblk)
            def _():
                wait_read(slot)

            @pl.when(bb < nfull)
            def _():
                transpose(slot, 8)
                pltpu.async_copy(out_v.at[slot], out_hbm.at[pl.ds(bb * 64, 64)],
                                 osem[slot])

            @pl.when(bb == nfull)
            def _():
                transpose(slot, 4)
                pltpu.async_copy(out_v.at[slot, pl.ds(0, 32)],
                                 out_hbm.at[pl.ds(bb * 64, 32)], osem[slot])
                pltpu.make_async_copy(out_v.at[slot, pl.ds(0, 32)],
                                      out_hbm.at[pl.ds(0, 32)],
                                      osem[slot]).wait()

        fire_read(wid, 0)
        nouter = (niter + 1) // 2

        def outer(g, carry):
            for par in range(2):
                step(g * 2 + par, par)
            return carry

        lax.fori_loop(0, nouter, outer, 0)
        last_bb = (2 * nouter - 2) * NW + wid
        @pl.when(last_bb < nfull)
        def _():
            pltpu.make_async_copy(out_v.at[0, pl.ds(0, 64)],
                                  out_hbm.at[pl.ds(0, 64)], osem[0]).wait()

    return pack_k(weights_t)


def kernel(token_ids, weights):
    B, S = token_ids.shape
    V, D = weights.shape
    TBLK = B // 128
    assert TBLK == NW and D == 64 and S % SB == 0
    idx_t = jnp.swapaxes(token_ids, 0, 1).astype(jnp.int32)
    table = _pack_table(jnp.swapaxes(weights, 0, 1)).reshape(V, D)

    @functools.partial(
        pl.kernel,
        mesh=plsc.VectorSubcoreMesh(core_axis_name="c", subcore_axis_name="s"),
        out_type=jax.ShapeDtypeStruct((S * 8 * TBLK, 8, 128), jnp.float32),
        compiler_params=pltpu.CompilerParams(use_tc_tiling_on_sc=False,
                                             needs_layout_passes=False),
        scratch_types=[
            pltpu.VMEM((S, 128), jnp.int32),
            pltpu.VMEM((SB, 128, D), jnp.float32),
            pltpu.VMEM((SB, D, 128), jnp.float32),
            [pltpu.SemaphoreType.DMA] * SB,
            [pltpu.SemaphoreType.DMA] * SB,
        ],
    )
    def gather_k(idx_hbm, table_hbm, out_hbm, idx_v, rows_v, tile_v, gsem, psem):
        wid = lax.axis_index("s") * NC + lax.axis_index("c")
        pltpu.sync_copy(idx_hbm.at[:, pl.ds(wid * 128, 128)], idx_v)

        iota = lax.iota(jnp.int32, LN)
        rvecs = [(iota + r) % LN for r in range(LN)]

        def fire_gather(m, slot):
            pltpu.async_copy(table_hbm.at[idx_v.at[m]], rows_v.at[slot],
                             gsem[slot])

        def wait_gather(slot):
            pltpu.make_async_copy(table_hbm.at[idx_v.at[0]], rows_v.at[slot],
                                  gsem[slot]).wait()

        def fire_puts(s, slot):
            for tr in range(8):
                pltpu.async_copy(tile_v.at[slot, pl.ds(tr * 8, 8)],
                                 out_hbm.at[(s * 8 + tr) * TBLK + wid],
                                 psem[slot])

        def wait_puts(slot):
            for _ in range(8):
                pltpu.make_async_copy(tile_v.at[slot, pl.ds(0, 8)],
                                      out_hbm.at[0], psem[slot]).wait()

        def transpose(slot):
            rows_2d = rows_v.at[slot]
            tile_2d = tile_v.at[slot]

            @plsc.parallel_loop(0, 128 // LN, unroll=2)
            def bt_body(bt):
                tok = iota + bt * LN
                for k in range(D // LN):
                    feats = [rvecs[r] + (k * LN) for r in range(LN)]
                    vals = [plsc.load_gather(rows_2d, [tok, feats[r]])
                            for r in range(LN)]
                    for r in range(LN):
                        plsc.store_scatter(tile_2d, [feats[r], tok], vals[r])

        def step(j, b):
            m = j + PF

            @pl.when(m < S)
            def _():
                fire_gather(m, (b + PF) % SB)

            wait_gather(b)

            @pl.when(j >= SB)
            def _():
                wait_puts(b)

            transpose(b)
            fire_puts(j, b)

        for m in range(PF):
            fire_gather(m, m)

        def outer(g, carry):
            for b in range(SB):
                step(g * SB + b, b)
            return carry

        lax.fori_loop(0, S // SB, outer, 0)

        for b in range(SB):
            wait_puts(b)

    out = gather_k(idx_t, table)
    o5 = out.reshape(S, 8, TBLK, 8, 128)
    return o5.transpose(2, 4, 0, 1, 3).reshape(B, S, D)


# speedup vs baseline: 1.7824x; 1.7824x over previous
import functools

import jax
import jax.numpy as jnp
from jax import lax
from jax.experimental import pallas as pl
from jax.experimental.pallas import tpu as pltpu
from jax.experimental.pallas import tpu_sc as plsc

NC = 2
NS = 16
NW = NC * NS
LN = 16
SB = 5
PF = 4


def _pack_table(weights_t):
    D, V = weights_t.shape
    nblk = (V + 127) // 128
    nfull = V // 128

    @functools.partial(
        pl.kernel,
        mesh=plsc.VectorSubcoreMesh(core_axis_name="c", subcore_axis_name="s"),
        out_type=jax.ShapeDtypeStruct((V // 2, 128), jnp.float32),
        compiler_params=pltpu.CompilerParams(use_tc_tiling_on_sc=True,
                                             needs_layout_passes=False),
        scratch_types=[
            pltpu.VMEM((2, D, 128), jnp.float32),
            pltpu.VMEM((2, D, 128), jnp.float32),
            [pltpu.SemaphoreType.DMA] * 2,
            [pltpu.SemaphoreType.DMA] * 2,
        ],
    )
    def pack_k(w_hbm, out_hbm, in_v, out_v, isem, osem):
        wid = lax.axis_index("s") * NC + lax.axis_index("c")
        iota = lax.iota(jnp.int32, LN)
        rvecs = [(iota + r) % LN for r in range(LN)]
        half = (iota % 2) * D
        niter = (nblk + NW - 1) // NW

        def fire_read(bb, slot):
            pltpu.async_copy(w_hbm.at[:, pl.ds(bb * 128, 128)], in_v.at[slot],
                             isem[slot])

        def wait_read(slot):
            pltpu.make_async_copy(w_hbm.at[:, pl.ds(0, 128)], in_v.at[slot],
                                  isem[slot]).wait()

        def transpose(slot, nbt):
            src = in_v.at[slot]
            dst = out_v.at[slot]

            def bt_body(bt, carry):
                vc = iota + bt * LN
                row = lax.shift_right_logical(vc, 1)
                for k in range(D // LN):
                    feats = [rvecs[r] + (k * LN) for r in range(LN)]
                    vals = [plsc.load_gather(src, [feats[r], vc])
                            for r in range(LN)]
                    for r in range(LN):
                        plsc.store_scatter(dst, [row, half + feats[r]],
                                           vals[r])
                return carry

            lax.fori_loop(0, nbt, bt_body, 0)

        def step(i, slot):
            bb = i * NW + wid

            nxt = bb + NW
            @pl.when(nxt < nblk)
            def _():
                fire_read(nxt, (slot + 1) % 2)

            @pl.when((i >= 2) & (bb - 2 * NW < nfull))
            def _():
                pltpu.make_async_copy(out_v.at[slot, pl.ds(0, 64)],
                                      out_hbm.at[pl.ds(0, 64)],
                                      osem[slot]).wait()

            @pl.when(bb < nblk)
            def _():
                wait_read(slot)

            @pl.when(bb < nfull)
            def _():
                transpose(slot, 8)
                pltpu.async_copy(out_v.at[slot], out_hbm.at[pl.ds(bb * 64, 64)],
                                 osem[slot])

            @pl.when(bb == nfull)
            def _():
                transpose(slot, 4)
                pltpu.async_copy(out_v.at[slot, pl.ds(0, 32)],
                                 out_hbm.at[pl.ds(bb * 64, 32)], osem[slot])
                pltpu.make_async_copy(out_v.at[slot, pl.ds(0, 32)],
                                      out_hbm.at[pl.ds(0, 32)],
                                      osem[slot]).wait()

        fire_read(wid, 0)
        nouter = (niter + 1) // 2

        def outer(g, carry):
            for par in range(2):
                step(g * 2 + par, par)
            return carry

        lax.fori_loop(0, nouter, outer, 0)
        last_bb = (2 * nouter - 2) * NW + wid
        @pl.when(last_bb < nfull)
        def _():
            pltpu.make_async_copy(out_v.at[0, pl.ds(0, 64)],
                                  out_hbm.at[pl.ds(0, 64)], osem[0]).wait()

    return pack_k(weights_t)


def kernel(token_ids, weights):
    B, S = token_ids.shape
    V, D = weights.shape
    TBLK = B // 128
    assert TBLK == NW and D == 64 and S % SB == 0
    idx_t = jnp.swapaxes(token_ids, 0, 1).astype(jnp.int32)
    table = _pack_table(jnp.swapaxes(weights, 0, 1)).reshape(V, D)

    @functools.partial(
        pl.kernel,
        mesh=plsc.VectorSubcoreMesh(core_axis_name="c", subcore_axis_name="s"),
        out_type=jax.ShapeDtypeStruct((S * 8 * TBLK, 8, 128), jnp.float32),
        compiler_params=pltpu.CompilerParams(use_tc_tiling_on_sc=False,
                                             needs_layout_passes=False),
        scratch_types=[
            pltpu.VMEM((S, 128), jnp.int32),
            pltpu.VMEM((SB, 128, D), jnp.float32),
            pltpu.VMEM((SB, D, 128), jnp.float32),
            [pltpu.SemaphoreType.DMA] * SB,
            [pltpu.SemaphoreType.DMA] * SB,
        ],
    )
    def gather_k(idx_hbm, table_hbm, out_hbm, idx_v, rows_v, tile_v, gsem, psem):
        wid = lax.axis_index("s") * NC + lax.axis_index("c")
        pltpu.sync_copy(idx_hbm.at[:, pl.ds(wid * 128, 128)], idx_v)

        iota = lax.iota(jnp.int32, LN)
        rvecs = [(iota + r) % LN for r in range(LN)]

        def fire_gather(m, slot):
            pltpu.async_copy(table_hbm.at[idx_v.at[m]], rows_v.at[slot],
                             gsem[slot])

        def wait_gather(slot):
            pltpu.make_async_copy(table_hbm.at[idx_v.at[0]], rows_v.at[slot],
                                  gsem[slot]).wait()

        def fire_puts(s, slot):
            for tr in range(8):
                pltpu.async_copy(tile_v.at[slot, pl.ds(tr * 8, 8)],
                                 out_hbm.at[(s * 8 + tr) * TBLK + wid],
                                 psem[slot])

        def wait_puts(slot):
            for _ in range(8):
                pltpu.make_async_copy(tile_v.at[slot, pl.ds(0, 8)],
                                      out_hbm.at[0], psem[slot]).wait()

        def transpose(slot):
            rows_2d = rows_v.at[slot]
            tile_2d = tile_v.at[slot]

            def bt_body(bt, carry):
                tok = iota + bt * LN
                for k in range(D // LN):
                    feats = [rvecs[r] + (k * LN) for r in range(LN)]
                    vals = [plsc.load_gather(rows_2d, [tok, feats[r]])
                            for r in range(LN)]
                    for r in range(LN):
                        plsc.store_scatter(tile_2d, [feats[r], tok], vals[r])
                return carry

            lax.fori_loop(0, 128 // LN, bt_body, 0)

        def step(j, b):
            m = j + PF

            @pl.when(m < S)
            def _():
                fire_gather(m, (b + PF) % SB)

            wait_gather(b)

            @pl.when(j >= SB)
            def _():
                wait_puts(b)

            transpose(b)
            fire_puts(j, b)

        for m in range(PF):
            fire_gather(m, m)

        def outer(g, carry):
            for b in range(SB):
                step(g * SB + b, b)
            return carry

        lax.fori_loop(0, S // SB, outer, 0)

        for b in range(SB):
            wait_puts(b)

    out = gather_k(idx_t, table)
    o5 = out.reshape(S, 8, TBLK, 8, 128)
    return o5.transpose(2, 4, 0, 1, 3).reshape(B, S, D)


# speedup vs baseline: 1.8087x; 1.0148x over previous
import functools

import jax
import jax.numpy as jnp
from jax import lax
from jax.experimental import pallas as pl
from jax.experimental.pallas import tpu as pltpu
from jax.experimental.pallas import tpu_sc as plsc

NC = 2
NS = 16
NW = NC * NS
LN = 16
SB = 4
PF = 3


def _pack_table(weights_t):
    D, V = weights_t.shape
    nblk = (V + 127) // 128
    nfull = V // 128

    @functools.partial(
        pl.kernel,
        mesh=plsc.VectorSubcoreMesh(core_axis_name="c", subcore_axis_name="s"),
        out_type=jax.ShapeDtypeStruct((V // 2, 128), jnp.float32),
        compiler_params=pltpu.CompilerParams(use_tc_tiling_on_sc=True,
                                             needs_layout_passes=False),
        scratch_types=[
            pltpu.VMEM((2, D, 128), jnp.float32),
            pltpu.VMEM((2, D, 128), jnp.float32),
            [pltpu.SemaphoreType.DMA] * 2,
            [pltpu.SemaphoreType.DMA] * 2,
        ],
    )
    def pack_k(w_hbm, out_hbm, in_v, out_v, isem, osem):
        wid = lax.axis_index("s") * NC + lax.axis_index("c")
        iota = lax.iota(jnp.int32, LN)
        rvecs = [(iota + r) % LN for r in range(LN)]
        half = (iota % 2) * D
        niter = (nblk + NW - 1) // NW

        def fire_read(bb, slot):
            pltpu.async_copy(w_hbm.at[:, pl.ds(bb * 128, 128)], in_v.at[slot],
                             isem[slot])

        def wait_read(slot):
            pltpu.make_async_copy(w_hbm.at[:, pl.ds(0, 128)], in_v.at[slot],
                                  isem[slot]).wait()

        def transpose(slot, nbt):
            src = in_v.at[slot]
            dst = out_v.at[slot]

            def bt_body(bt, carry):
                vc = iota + bt * LN
                row = lax.shift_right_logical(vc, 1)
                for k in range(D // LN):
                    feats = [rvecs[r] + (k * LN) for r in range(LN)]
                    vals = [plsc.load_gather(src, [feats[r], vc])
                            for r in range(LN)]
                    for r in range(LN):
                        plsc.store_scatter(dst, [row, half + feats[r]],
                                           vals[r])
                return carry

            lax.fori_loop(0, nbt, bt_body, 0)

        def step(i, slot):
            bb = i * NW + wid

            nxt = bb + NW
            @pl.when(nxt < nblk)
            def _():
                fire_read(nxt, (slot + 1) % 2)

            @pl.when((i >= 2) & (bb - 2 * NW < nfull))
            def _():
                pltpu.make_async_copy(out_v.at[slot, pl.ds(0, 64)],
                                      out_hbm.at[pl.ds(0, 64)],
                                      osem[slot]).wait()

            @pl.when(bb < nblk)
            def _():
                wait_read(slot)

            @pl.when(bb < nfull)
            def _():
                transpose(slot, 8)
                pltpu.async_copy(out_v.at[slot], out_hbm.at[pl.ds(bb * 64, 64)],
                                 osem[slot])

            @pl.when(bb == nfull)
            def _():
                transpose(slot, 4)
                pltpu.async_copy(out_v.at[slot, pl.ds(0, 32)],
                                 out_hbm.at[pl.ds(bb * 64, 32)], osem[slot])
                pltpu.make_async_copy(out_v.at[slot, pl.ds(0, 32)],
                                      out_hbm.at[pl.ds(0, 32)],
                                      osem[slot]).wait()

        fire_read(wid, 0)
        nouter = (niter + 1) // 2

        def outer(g, carry):
            for par in range(2):
                step(g * 2 + par, par)
            return carry

        lax.fori_loop(0, nouter, outer, 0)
        last_bb = (2 * nouter - 2) * NW + wid
        @pl.when(last_bb < nfull)
        def _():
            pltpu.make_async_copy(out_v.at[0, pl.ds(0, 64)],
                                  out_hbm.at[pl.ds(0, 64)], osem[0]).wait()

    return pack_k(weights_t)


def kernel(token_ids, weights):
    B, S = token_ids.shape
    V, D = weights.shape
    TBLK = B // 128
    assert TBLK == NW and D == 64 and S % SB == 0
    idx_t = jnp.swapaxes(token_ids, 0, 1).astype(jnp.int32)
    table = _pack_table(jnp.swapaxes(weights, 0, 1)).reshape(V, D)

    @functools.partial(
        pl.kernel,
        mesh=plsc.VectorSubcoreMesh(core_axis_name="c", subcore_axis_name="s"),
        out_type=jax.ShapeDtypeStruct((S * 8 * TBLK, 8, 128), jnp.float32),
        compiler_params=pltpu.CompilerParams(use_tc_tiling_on_sc=False,
                                             needs_layout_passes=False),
        scratch_types=[
            pltpu.VMEM((S, 128), jnp.int32),
            pltpu.VMEM((SB, 128, D), jnp.float32),
            pltpu.VMEM((SB, D, 128), jnp.float32),
            [pltpu.SemaphoreType.DMA] * SB,
            [pltpu.SemaphoreType.DMA] * SB,
        ],
    )
    def gather_k(idx_hbm, table_hbm, out_hbm, idx_v, rows_v, tile_v, gsem, psem):
        wid = lax.axis_index("s") * NC + lax.axis_index("c")
        pltpu.sync_copy(idx_hbm.at[:, pl.ds(wid * 128, 128)], idx_v)

        iota = lax.iota(jnp.int32, LN)
        rvecs = [(iota + r) % LN for r in range(LN)]

        def fire_gather(m, slot):
            pltpu.async_copy(table_hbm.at[idx_v.at[m]], rows_v.at[slot],
                             gsem[slot])

        def wait_gather(slot):
            pltpu.make_async_copy(table_hbm.at[idx_v.at[0]], rows_v.at[slot],
                                  gsem[slot]).wait()

        def fire_puts(s, slot):
            for tr in range(8):
                pltpu.async_copy(tile_v.at[slot, pl.ds(tr * 8, 8)],
                                 out_hbm.at[(s * 8 + tr) * TBLK + wid],
                                 psem[slot])

        def wait_puts(slot):
            for _ in range(8):
                pltpu.make_async_copy(tile_v.at[slot, pl.ds(0, 8)],
                                      out_hbm.at[0], psem[slot]).wait()

        def transpose(slot):
            rows_2d = rows_v.at[slot]
            tile_2d = tile_v.at[slot]

            def bt_body(bt, carry):
                tok = iota + bt * LN
                for k in range(D // LN):
                    feats = [rvecs[r] + (k * LN) for r in range(LN)]
                    vals = [plsc.load_gather(rows_2d, [tok, feats[r]])
                            for r in range(LN)]
                    for r in range(LN):
                        plsc.store_scatter(tile_2d, [feats[r], tok], vals[r])
                return carry

            lax.fori_loop(0, 128 // LN, bt_body, 0)

        def step(j, b):
            m = j + PF

            @pl.when(m < S)
            def _():
                fire_gather(m, (b + PF) % SB)

            wait_gather(b)

            @pl.when(j >= SB)
            def _():
                wait_puts(b)

            transpose(b)
            fire_puts(j, b)

        for m in range(PF):
            fire_gather(m, m)

        def outer(g, carry):
            for b in range(SB):
                step(g * SB + b, b)
            return carry

        lax.fori_loop(0, S // SB, outer, 0)

        for b in range(SB):
            wait_puts(b)

    out = gather_k(idx_t, table)
    o5 = out.reshape(S, 8, TBLK, 8, 128)
    return o5.transpose(2, 4, 0, 1, 3).reshape(B, S, D)
